# Initial kernel scaffold; baseline (speedup 1.0000x reference)
import jax
import jax.numpy as jnp
from jax.experimental import pallas as pl


def kernel(x, edge_index, W, attn_j, attn_i, bias):
    raise NotImplementedError("write your pallas kernel here")


# speedup vs baseline: 1.5413x; 1.5413x over previous
import jax
import jax.numpy as jnp
from jax.experimental import pallas as pl

_N = 10000
_D = 128
_E_PAD = 331776


def _tc_proj_body(x_ref, wt_ref, aj_ref, ai_ref, h_ref, sj_ref, si_ref):
    h = jnp.dot(x_ref[...], wt_ref[...], preferred_element_type=jnp.float32)
    h_ref[...] = h
    sj_ref[...] = jnp.sum(h * aj_ref[...], axis=-1, keepdims=True)
    si_ref[...] = jnp.sum(h * ai_ref[...], axis=-1, keepdims=True)


def _tc_proj(x, wt, aj, ai):
    blk = 1000
    grid = _N // blk
    return pl.pallas_call(
        _tc_proj_body,
        grid=(grid,),
        in_specs=[
            pl.BlockSpec((blk, _D), lambda i: (i, 0)),
            pl.BlockSpec((_D, _D), lambda i: (0, 0)),
            pl.BlockSpec((1, _D), lambda i: (0, 0)),
            pl.BlockSpec((1, _D), lambda i: (0, 0)),
        ],
        out_specs=[
            pl.BlockSpec((blk, _D), lambda i: (i, 0)),
            pl.BlockSpec((blk, 1), lambda i: (i, 0)),
            pl.BlockSpec((blk, 1), lambda i: (i, 0)),
        ],
        out_shape=[
            jax.ShapeDtypeStruct((_N, _D), jnp.float32),
            jax.ShapeDtypeStruct((_N, 1), jnp.float32),
            jax.ShapeDtypeStruct((_N, 1), jnp.float32),
        ],
    )(x, wt, aj, ai)


def _edge_body(aj_ref, ai_ref, hg_ref, msg_ref, w_ref):
    a = aj_ref[...] + ai_ref[...]
    a = jnp.where(a >= 0.0, a, a * 0.2)
    w = jnp.exp(a)
    w_ref[...] = w
    msg_ref[...] = hg_ref[...] * w


def _edge_stage(aj, ai, hg):
    blk = 1296
    grid = _E_PAD // blk
    return pl.pallas_call(
        _edge_body,
        grid=(grid,),
        in_specs=[
            pl.BlockSpec((blk, 1), lambda i: (i, 0)),
            pl.BlockSpec((blk, 1), lambda i: (i, 0)),
            pl.BlockSpec((blk, _D), lambda i: (i, 0)),
        ],
        out_specs=[
            pl.BlockSpec((blk, _D), lambda i: (i, 0)),
            pl.BlockSpec((blk, 1), lambda i: (i, 0)),
        ],
        out_shape=[
            jax.ShapeDtypeStruct((_E_PAD, _D), jnp.float32),
            jax.ShapeDtypeStruct((_E_PAD, 1), jnp.float32),
        ],
    )(aj, ai, hg)


def _tc_combine_body(msg_ref, wsum_ref, bias_ref, out_ref):
    out_ref[...] = msg_ref[...] / wsum_ref[...] + bias_ref[...]


def _tc_combine(msg, wsum, bias2d):
    blk = 1000
    grid = _N // blk
    return pl.pallas_call(
        _tc_combine_body,
        grid=(grid,),
        in_specs=[
            pl.BlockSpec((blk, _D), lambda i: (i, 0)),
            pl.BlockSpec((blk, 1), lambda i: (i, 0)),
            pl.BlockSpec((1, _D), lambda i: (0, 0)),
        ],
        out_specs=pl.BlockSpec((blk, _D), lambda i: (i, 0)),
        out_shape=jax.ShapeDtypeStruct((_N, _D), jnp.float32),
    )(msg, wsum, bias2d)


def kernel(x, edge_index, W, attn_j, attn_i, bias):
    n = x.shape[0]
    e = edge_index.shape[1]

    h, sj, si = _tc_proj(
        x,
        W.T,
        attn_j.reshape(1, -1).astype(jnp.float32),
        attn_i.reshape(1, -1).astype(jnp.float32),
    )
    sj = sj[:, 0]
    si = si[:, 0]

    src = edge_index[0]
    dst = edge_index[1]
    mask = src != dst
    n_keep = mask.sum()
    cs = jnp.cumsum(mask.astype(jnp.int32))
    q = jnp.arange(n, dtype=jnp.int32)
    oe = jnp.clip(jnp.searchsorted(cs, q + 1, side="left"), 0, e - 1)
    cej = jnp.where(q < n_keep, src[oe], q - n_keep)
    cei = jnp.where(q < n_keep, dst[oe], q - n_keep)
    tj = sj[cej]
    ti = si[cei]

    loops = jnp.arange(n, dtype=jnp.int32)
    pad = _E_PAD - e - n
    src_all = jnp.concatenate([src, loops, jnp.zeros((pad,), jnp.int32)])
    dst_all = jnp.concatenate(
        [jnp.where(mask, dst, n), loops, jnp.full((pad,), n, jnp.int32)]
    )

    aj = tj[src_all].reshape(-1, 1)
    ai = ti[jnp.minimum(dst_all, n - 1)].reshape(-1, 1)
    hg = h[src_all]

    msg, w = _edge_stage(aj, ai, hg)

    wsum = jax.ops.segment_sum(w[:, 0], dst_all, num_segments=n + 1)[:n]
    msgsum = jax.ops.segment_sum(msg, dst_all, num_segments=n + 1)[:n]

    return _tc_combine(msgsum, wsum.reshape(-1, 1), bias.reshape(1, -1))
